# baseline (speedup 1.0000x reference)
import functools
import math

import jax
import jax.numpy as jnp
from jax import lax
from jax.experimental import pallas as pl
from jax.experimental.pallas import tpu as pltpu
from jax.experimental.pallas import tpu_sc as plsc

NUM_SC_CORES = 2
NUM_SC_SUBCORES = 16
NUM_TILES = NUM_SC_CORES * NUM_SC_SUBCORES

GATHER_CHUNK = 64


def _sc_gather(token_table, flat_idx, n_rows, d):
    rows_per_tile = n_rows // NUM_TILES
    n_chunks = rows_per_tile // GATHER_CHUNK
    mesh = plsc.VectorSubcoreMesh(core_axis_name="c", subcore_axis_name="s")

    @functools.partial(
        pl.kernel,
        out_type=jax.ShapeDtypeStruct((n_rows, d), jnp.float32),
        mesh=mesh,
        scratch_types=[
            pltpu.VMEM((rows_per_tile,), jnp.int32),
            pltpu.VMEM((GATHER_CHUNK, d), jnp.float32),
            pltpu.VMEM((GATHER_CHUNK, d), jnp.float32),
            pltpu.SemaphoreType.DMA,
            pltpu.SemaphoreType.DMA,
            pltpu.SemaphoreType.DMA,
            pltpu.SemaphoreType.DMA,
        ],
    )
    def sc_kernel(
        table_hbm, idx_hbm, out_hbm, idx_v, rows0, rows1, gs0, gs1, ws0, ws1
    ):
        wid = lax.axis_index("s") * NUM_SC_CORES + lax.axis_index("c")
        base = wid * rows_per_tile
        pltpu.sync_copy(idx_hbm.at[pl.ds(base, rows_per_tile)], idx_v)

        def g_desc(c, buf, sem):
            return pltpu.make_async_copy(
                table_hbm.at[idx_v.at[pl.ds(c * GATHER_CHUNK, GATHER_CHUNK)]],
                buf,
                sem,
            )

        def w_desc(c, buf, sem):
            return pltpu.make_async_copy(
                buf, out_hbm.at[pl.ds(base + c * GATHER_CHUNK, GATHER_CHUNK)], sem
            )

        g_desc(0, rows0, gs0).start()
        g_desc(1, rows1, gs1).start()

        @pl.loop(0, n_chunks, step=2)
        def _(c):
            g_desc(c, rows0, gs0).wait()
            w_desc(c, rows0, ws0).start()
            g_desc(c + 1, rows1, gs1).wait()
            w_desc(c + 1, rows1, ws1).start()
            w_desc(c, rows0, ws0).wait()

            @pl.when(c + 2 < n_chunks)
            def _():
                g_desc(c + 2, rows0, gs0).start()

            w_desc(c + 1, rows1, ws1).wait()

            @pl.when(c + 3 < n_chunks)
            def _():
                g_desc(c + 3, rows1, gs1).start()

    return sc_kernel(token_table, flat_idx)


def _pe_gen_body(pe_ref, theta_ref, sinb_ref, cosb_ref, *, bs, d):
    i = pl.program_id(0)

    @pl.when(i == 0)
    def _():
        col = lax.broadcasted_iota(jnp.int32, (16, d), 1).astype(jnp.float32)
        theta = jnp.exp((2.0 * jnp.floor(col * 0.5)) * (-math.log(10000.0) / d))
        theta_ref[...] = theta
        s_lo = lax.broadcasted_iota(jnp.int32, (16, d), 0).astype(jnp.float32)
        ang_b = s_lo * theta
        sinb_ref[...] = jnp.sin(ang_b)
        cosb_ref[...] = jnp.sin(ang_b + 0.5 * math.pi)

    nh = bs // 16
    theta_row = theta_ref[0:1, :]
    s_hi = (i * bs + 16 * lax.broadcasted_iota(jnp.int32, (nh, d), 0)).astype(
        jnp.float32
    )
    ang_a = s_hi * theta_row
    sin_a = jnp.sin(ang_a).reshape(nh, 1, d)
    cos_a = jnp.sin(ang_a + 0.5 * math.pi).reshape(nh, 1, d)
    sin_b = sinb_ref[...].reshape(1, 16, d)
    cos_b = cosb_ref[...].reshape(1, 16, d)
    is_even = (lax.broadcasted_iota(jnp.int32, (1, 16, d), 2) % 2) == 0
    pe3 = jnp.where(
        is_even,
        sin_a * cos_b + cos_a * sin_b,
        cos_a * cos_b - sin_a * sin_b,
    )
    pe_ref[...] = pe3.reshape(bs, d)


def _pe_gen(s, d, bs):
    return pl.pallas_call(
        functools.partial(_pe_gen_body, bs=bs, d=d),
        grid=(s // bs,),
        in_specs=[],
        out_specs=pl.BlockSpec((bs, d), lambda i: (i, 0)),
        out_shape=jax.ShapeDtypeStruct((s, d), jnp.float32),
        scratch_shapes=[
            pltpu.VMEM((16, d), jnp.float32),
            pltpu.VMEM((16, d), jnp.float32),
            pltpu.VMEM((16, d), jnp.float32),
        ],
        compiler_params=pltpu.CompilerParams(dimension_semantics=("arbitrary",)),
    )()


def _add_block(seg_ids_ref, tok_ref, pe_ref, seg_table_ref, out_ref):
    ids = seg_ids_ref[0, 0, :][:, None]
    r0 = seg_table_ref[0, :][None, :]
    r1 = seg_table_ref[1, :][None, :]
    r2 = seg_table_ref[2, :][None, :]
    seg = jnp.where(ids == 2, r2, jnp.where(ids == 1, r1, r0))
    out_ref[0] = tok_ref[0] + pe_ref[...] + seg


def _tc_add_first_body(seg_ids_ref, tok_ref, pe_ref, seg_table_ref, out_ref):
    _add_block(seg_ids_ref, tok_ref, pe_ref, seg_table_ref, out_ref)


def _tc_add_alias_body(prev_ref, seg_ids_ref, tok_ref, pe_ref, seg_table_ref, out_ref):
    del prev_ref
    _add_block(seg_ids_ref, tok_ref, pe_ref, seg_table_ref, out_ref)


def _tc_add_slab(prev, tok_slab, seg_ids3, pe, segment_table, batch, b_off, bs):
    nb, s, d = tok_slab.shape
    grid = (s // bs, nb)
    specs = [
        pl.BlockSpec((1, 1, bs), lambda i, b: (b, 0, i)),
        pl.BlockSpec((1, bs, d), lambda i, b: (b, i, 0)),
        pl.BlockSpec((bs, d), lambda i, b: (i, 0)),
        pl.BlockSpec((3, d), lambda i, b: (0, 0)),
    ]
    out_spec = pl.BlockSpec((1, bs, d), lambda i, b: (b + b_off, i, 0))
    out_shape = jax.ShapeDtypeStruct((batch, s, d), jnp.float32)
    if prev is None:
        return pl.pallas_call(
            _tc_add_first_body,
            grid=grid,
            in_specs=specs,
            out_specs=out_spec,
            out_shape=out_shape,
            compiler_params=pltpu.CompilerParams(
                dimension_semantics=("arbitrary", "arbitrary")
            ),
        )(seg_ids3, tok_slab, pe, segment_table)
    return pl.pallas_call(
        _tc_add_alias_body,
        grid=grid,
        in_specs=[pl.BlockSpec(memory_space=pl.ANY)] + specs,
        out_specs=out_spec,
        out_shape=out_shape,
        input_output_aliases={0: 0},
        compiler_params=pltpu.CompilerParams(
            dimension_semantics=("arbitrary", "arbitrary")
        ),
    )(prev, seg_ids3, tok_slab, pe, segment_table)


def kernel(sequence, segment_ids, token_table, segment_table):
    batch, s = sequence.shape
    vocab, d = token_table.shape
    n = batch * s
    half_b = batch // 2
    half_n = n // 2

    pe = _pe_gen(s, d, bs=512)
    seq_flat = sequence.reshape(n)
    seg3 = segment_ids.reshape(batch, 1, s)

    tok0 = _sc_gather(token_table, seq_flat[:half_n], half_n, d)
    tok1 = _sc_gather(token_table, seq_flat[half_n:], half_n, d)

    out = _tc_add_slab(
        None, tok0.reshape(half_b, s, d), seg3[:half_b], pe, segment_table,
        batch, b_off=0, bs=512,
    )
    out = _tc_add_slab(
        out, tok1.reshape(half_b, s, d), seg3[half_b:], pe, segment_table,
        batch, b_off=half_b, bs=512,
    )
    return out


# speedup vs baseline: 1.0502x; 1.0502x over previous
import functools
import math

import jax
import jax.numpy as jnp
from jax import lax
from jax.experimental import pallas as pl
from jax.experimental.pallas import tpu as pltpu
from jax.experimental.pallas import tpu_sc as plsc

NUM_SC_CORES = 2
NUM_SC_SUBCORES = 16
NUM_TILES = NUM_SC_CORES * NUM_SC_SUBCORES

GATHER_CHUNK = 64


def _sc_gather(token_table, flat_idx, n_rows, d):
    rows_per_tile = n_rows // NUM_TILES
    n_chunks = rows_per_tile // GATHER_CHUNK
    mesh = plsc.VectorSubcoreMesh(core_axis_name="c", subcore_axis_name="s")

    @functools.partial(
        pl.kernel,
        out_type=jax.ShapeDtypeStruct((n_rows, d), jnp.float32),
        mesh=mesh,
        scratch_types=[
            pltpu.VMEM((rows_per_tile,), jnp.int32),
            pltpu.VMEM((GATHER_CHUNK, d), jnp.float32),
            pltpu.VMEM((GATHER_CHUNK, d), jnp.float32),
            pltpu.SemaphoreType.DMA,
            pltpu.SemaphoreType.DMA,
            pltpu.SemaphoreType.DMA,
            pltpu.SemaphoreType.DMA,
        ],
    )
    def sc_kernel(
        table_hbm, idx_hbm, out_hbm, idx_v, rows0, rows1, gs0, gs1, ws0, ws1
    ):
        wid = lax.axis_index("s") * NUM_SC_CORES + lax.axis_index("c")
        base = wid * rows_per_tile
        pltpu.sync_copy(idx_hbm.at[pl.ds(base, rows_per_tile)], idx_v)

        def g_desc(c, buf, sem):
            return pltpu.make_async_copy(
                table_hbm.at[idx_v.at[pl.ds(c * GATHER_CHUNK, GATHER_CHUNK)]],
                buf,
                sem,
            )

        def w_desc(c, buf, sem):
            return pltpu.make_async_copy(
                buf, out_hbm.at[pl.ds(base + c * GATHER_CHUNK, GATHER_CHUNK)], sem
            )

        g_desc(0, rows0, gs0).start()
        g_desc(1, rows1, gs1).start()

        @pl.loop(0, n_chunks, step=2)
        def _(c):
            g_desc(c, rows0, gs0).wait()
            w_desc(c, rows0, ws0).start()
            g_desc(c + 1, rows1, gs1).wait()
            w_desc(c + 1, rows1, ws1).start()
            w_desc(c, rows0, ws0).wait()

            @pl.when(c + 2 < n_chunks)
            def _():
                g_desc(c + 2, rows0, gs0).start()

            w_desc(c + 1, rows1, ws1).wait()

            @pl.when(c + 3 < n_chunks)
            def _():
                g_desc(c + 3, rows1, gs1).start()

    return sc_kernel(token_table, flat_idx)


def _pe_gen_body(pe_ref, theta_ref, sinb_ref, cosb_ref, *, bs, d):
    i = pl.program_id(0)

    @pl.when(i == 0)
    def _():
        col = lax.broadcasted_iota(jnp.int32, (16, d), 1).astype(jnp.float32)
        theta = jnp.exp((2.0 * jnp.floor(col * 0.5)) * (-math.log(10000.0) / d))
        theta_ref[...] = theta
        s_lo = lax.broadcasted_iota(jnp.int32, (16, d), 0).astype(jnp.float32)
        ang_b = s_lo * theta
        sinb_ref[...] = jnp.sin(ang_b)
        cosb_ref[...] = jnp.sin(ang_b + 0.5 * math.pi)

    nh = bs // 16
    theta_row = theta_ref[0:1, :]
    s_hi = (i * bs + 16 * lax.broadcasted_iota(jnp.int32, (nh, d), 0)).astype(
        jnp.float32
    )
    ang_a = s_hi * theta_row
    sin_a = jnp.sin(ang_a).reshape(nh, 1, d)
    cos_a = jnp.sin(ang_a + 0.5 * math.pi).reshape(nh, 1, d)
    sin_b = sinb_ref[...].reshape(1, 16, d)
    cos_b = cosb_ref[...].reshape(1, 16, d)
    is_even = (lax.broadcasted_iota(jnp.int32, (1, 16, d), 2) % 2) == 0
    pe3 = jnp.where(
        is_even,
        sin_a * cos_b + cos_a * sin_b,
        cos_a * cos_b - sin_a * sin_b,
    )
    pe_ref[...] = pe3.reshape(bs, d)


def _pe_gen(s, d, bs):
    return pl.pallas_call(
        functools.partial(_pe_gen_body, bs=bs, d=d),
        grid=(s // bs,),
        in_specs=[],
        out_specs=pl.BlockSpec((bs, d), lambda i: (i, 0)),
        out_shape=jax.ShapeDtypeStruct((s, d), jnp.float32),
        scratch_shapes=[
            pltpu.VMEM((16, d), jnp.float32),
            pltpu.VMEM((16, d), jnp.float32),
            pltpu.VMEM((16, d), jnp.float32),
        ],
        compiler_params=pltpu.CompilerParams(dimension_semantics=("arbitrary",)),
    )()


def _add_block(seg_ids_ref, tok_ref, pe_ref, seg_table_ref, out_ref):
    ids = seg_ids_ref[0, 0, :][:, None]
    r0 = seg_table_ref[0, :][None, :]
    r1 = seg_table_ref[1, :][None, :]
    r2 = seg_table_ref[2, :][None, :]
    seg = jnp.where(ids == 2, r2, jnp.where(ids == 1, r1, r0))
    out_ref[0] = tok_ref[0] + pe_ref[...] + seg


def _tc_add_first_body(seg_ids_ref, tok_ref, pe_ref, seg_table_ref, out_ref):
    _add_block(seg_ids_ref, tok_ref, pe_ref, seg_table_ref, out_ref)


def _tc_add_alias_body(prev_ref, seg_ids_ref, tok_ref, pe_ref, seg_table_ref, out_ref):
    del prev_ref
    _add_block(seg_ids_ref, tok_ref, pe_ref, seg_table_ref, out_ref)


def _tc_add_slab(prev, tok_slab, seg_ids3, pe, segment_table, s_total, s_off, bs):
    nb, s_slab, d = tok_slab.shape
    grid = (s_slab // bs, nb)
    i_off = s_off // bs
    specs = [
        pl.BlockSpec((1, 1, bs), lambda i, b: (b, 0, i)),
        pl.BlockSpec((1, bs, d), lambda i, b: (b, i, 0)),
        pl.BlockSpec((bs, d), lambda i, b: (i + i_off, 0)),
        pl.BlockSpec((3, d), lambda i, b: (0, 0)),
    ]
    out_spec = pl.BlockSpec((1, bs, d), lambda i, b: (b, i + i_off, 0))
    out_shape = jax.ShapeDtypeStruct((nb, s_total, d), jnp.float32)
    if prev is None:
        return pl.pallas_call(
            _tc_add_first_body,
            grid=grid,
            in_specs=specs,
            out_specs=out_spec,
            out_shape=out_shape,
            compiler_params=pltpu.CompilerParams(
                dimension_semantics=("arbitrary", "arbitrary")
            ),
        )(seg_ids3, tok_slab, pe, segment_table)
    return pl.pallas_call(
        _tc_add_alias_body,
        grid=grid,
        in_specs=[pl.BlockSpec(memory_space=pl.ANY)] + specs,
        out_specs=out_spec,
        out_shape=out_shape,
        input_output_aliases={0: 0},
        compiler_params=pltpu.CompilerParams(
            dimension_semantics=("arbitrary", "arbitrary")
        ),
    )(prev, seg_ids3, tok_slab, pe, segment_table)


def kernel(sequence, segment_ids, token_table, segment_table):
    batch, s = sequence.shape
    vocab, d = token_table.shape
    half_s = s // 2
    half_n = batch * half_s

    pe = _pe_gen(s, d, bs=512)
    seg3 = segment_ids.reshape(batch, 1, s)

    idx0 = sequence[:, :half_s].reshape(half_n)
    idx1 = sequence[:, half_s:].reshape(half_n)
    tok0 = _sc_gather(token_table, idx0, half_n, d)
    tok1 = _sc_gather(token_table, idx1, half_n, d)

    out = _tc_add_slab(
        None, tok0.reshape(batch, half_s, d), seg3[:, :, :half_s], pe,
        segment_table, s, s_off=0, bs=1024,
    )
    out = _tc_add_slab(
        out, tok1.reshape(batch, half_s, d), seg3[:, :, half_s:], pe,
        segment_table, s, s_off=half_s, bs=1024,
    )
    return out


# speedup vs baseline: 1.0954x; 1.0430x over previous
import functools
import math

import jax
import jax.numpy as jnp
from jax import lax
from jax.experimental import pallas as pl
from jax.experimental.pallas import tpu as pltpu
from jax.experimental.pallas import tpu_sc as plsc

NUM_SC_CORES = 2
NUM_SC_SUBCORES = 16
NUM_TILES = NUM_SC_CORES * NUM_SC_SUBCORES

GATHER_CHUNK = 64


def _sc_gather(token_table, flat_idx, n_rows, d):
    rows_per_tile = n_rows // NUM_TILES
    n_chunks = rows_per_tile // GATHER_CHUNK
    mesh = plsc.VectorSubcoreMesh(core_axis_name="c", subcore_axis_name="s")

    @functools.partial(
        pl.kernel,
        out_type=jax.ShapeDtypeStruct((n_rows, d), jnp.float32),
        mesh=mesh,
        scratch_types=[
            pltpu.VMEM((rows_per_tile,), jnp.int32),
            pltpu.VMEM((GATHER_CHUNK, d), jnp.float32),
            pltpu.VMEM((GATHER_CHUNK, d), jnp.float32),
            pltpu.SemaphoreType.DMA,
            pltpu.SemaphoreType.DMA,
            pltpu.SemaphoreType.DMA,
            pltpu.SemaphoreType.DMA,
        ],
    )
    def sc_kernel(
        table_hbm, idx_hbm, out_hbm, idx_v, rows0, rows1, gs0, gs1, ws0, ws1
    ):
        wid = lax.axis_index("s") * NUM_SC_CORES + lax.axis_index("c")
        base = wid * rows_per_tile
        pltpu.sync_copy(idx_hbm.at[pl.ds(base, rows_per_tile)], idx_v)

        def g_desc(c, buf, sem):
            return pltpu.make_async_copy(
                table_hbm.at[idx_v.at[pl.ds(c * GATHER_CHUNK, GATHER_CHUNK)]],
                buf,
                sem,
            )

        def w_desc(c, buf, sem):
            return pltpu.make_async_copy(
                buf, out_hbm.at[pl.ds(base + c * GATHER_CHUNK, GATHER_CHUNK)], sem
            )

        g_desc(0, rows0, gs0).start()
        g_desc(1, rows1, gs1).start()

        @pl.loop(0, n_chunks, step=2)
        def _(c):
            g_desc(c, rows0, gs0).wait()
            w_desc(c, rows0, ws0).start()
            g_desc(c + 1, rows1, gs1).wait()
            w_desc(c + 1, rows1, ws1).start()
            w_desc(c, rows0, ws0).wait()

            @pl.when(c + 2 < n_chunks)
            def _():
                g_desc(c + 2, rows0, gs0).start()

            w_desc(c + 1, rows1, ws1).wait()

            @pl.when(c + 3 < n_chunks)
            def _():
                g_desc(c + 3, rows1, gs1).start()

    return sc_kernel(token_table, flat_idx)


def _pe_gen_body(pe_ref, theta_ref, sinb_ref, cosb_ref, *, bs, d):
    i = pl.program_id(0)

    @pl.when(i == 0)
    def _():
        col = lax.broadcasted_iota(jnp.int32, (16, d), 1).astype(jnp.float32)
        theta = jnp.exp((2.0 * jnp.floor(col * 0.5)) * (-math.log(10000.0) / d))
        theta_ref[...] = theta
        s_lo = lax.broadcasted_iota(jnp.int32, (16, d), 0).astype(jnp.float32)
        ang_b = s_lo * theta
        sinb_ref[...] = jnp.sin(ang_b)
        cosb_ref[...] = jnp.sin(ang_b + 0.5 * math.pi)

    nh = bs // 16
    theta_row = theta_ref[0:1, :]
    s_hi = (i * bs + 16 * lax.broadcasted_iota(jnp.int32, (nh, d), 0)).astype(
        jnp.float32
    )
    ang_a = s_hi * theta_row
    sin_a = jnp.sin(ang_a).reshape(nh, 1, d)
    cos_a = jnp.sin(ang_a + 0.5 * math.pi).reshape(nh, 1, d)
    sin_b = sinb_ref[...].reshape(1, 16, d)
    cos_b = cosb_ref[...].reshape(1, 16, d)
    is_even = (lax.broadcasted_iota(jnp.int32, (1, 16, d), 2) % 2) == 0
    pe3 = jnp.where(
        is_even,
        sin_a * cos_b + cos_a * sin_b,
        cos_a * cos_b - sin_a * sin_b,
    )
    pe_ref[...] = pe3.reshape(bs, d)


def _pe_gen(s, d, bs):
    return pl.pallas_call(
        functools.partial(_pe_gen_body, bs=bs, d=d),
        grid=(s // bs,),
        in_specs=[],
        out_specs=pl.BlockSpec((bs, d), lambda i: (i, 0)),
        out_shape=jax.ShapeDtypeStruct((s, d), jnp.float32),
        scratch_shapes=[
            pltpu.VMEM((16, d), jnp.float32),
            pltpu.VMEM((16, d), jnp.float32),
            pltpu.VMEM((16, d), jnp.float32),
        ],
        compiler_params=pltpu.CompilerParams(dimension_semantics=("arbitrary",)),
    )()


def _add_block(seg_ids_ref, tok_ref, pe_ref, seg_table_ref, out_ref):
    ids = seg_ids_ref[0, 0, :][:, None]
    r0 = seg_table_ref[0, :][None, :]
    r1 = seg_table_ref[1, :][None, :]
    r2 = seg_table_ref[2, :][None, :]
    seg = jnp.where(ids == 2, r2, jnp.where(ids == 1, r1, r0))
    out_ref[0] = tok_ref[0] + pe_ref[...] + seg


def _tc_add_first_body(seg_ids_ref, tok_ref, pe_ref, seg_table_ref, out_ref):
    _add_block(seg_ids_ref, tok_ref, pe_ref, seg_table_ref, out_ref)


def _tc_add_alias_body(prev_ref, seg_ids_ref, tok_ref, pe_ref, seg_table_ref, out_ref):
    del prev_ref
    _add_block(seg_ids_ref, tok_ref, pe_ref, seg_table_ref, out_ref)


def _tc_add_slab(prev, tok_slab, seg_ids3, pe, segment_table, s_total, s_off, bs):
    nb, s_slab, d = tok_slab.shape
    grid = (s_slab // bs, nb)
    i_off = s_off // bs
    specs = [
        pl.BlockSpec((1, 1, bs), lambda i, b: (b, 0, i)),
        pl.BlockSpec((1, bs, d), lambda i, b: (b, i, 0)),
        pl.BlockSpec((bs, d), lambda i, b: (i + i_off, 0)),
        pl.BlockSpec((3, d), lambda i, b: (0, 0)),
    ]
    out_spec = pl.BlockSpec((1, bs, d), lambda i, b: (b, i + i_off, 0))
    out_shape = jax.ShapeDtypeStruct((nb, s_total, d), jnp.float32)
    if prev is None:
        return pl.pallas_call(
            _tc_add_first_body,
            grid=grid,
            in_specs=specs,
            out_specs=out_spec,
            out_shape=out_shape,
            compiler_params=pltpu.CompilerParams(
                dimension_semantics=("arbitrary", "arbitrary")
            ),
        )(seg_ids3, tok_slab, pe, segment_table)
    return pl.pallas_call(
        _tc_add_alias_body,
        grid=grid,
        in_specs=[pl.BlockSpec(memory_space=pl.ANY)] + specs,
        out_specs=out_spec,
        out_shape=out_shape,
        input_output_aliases={0: 0},
        compiler_params=pltpu.CompilerParams(
            dimension_semantics=("arbitrary", "arbitrary")
        ),
    )(prev, seg_ids3, tok_slab, pe, segment_table)


def kernel(sequence, segment_ids, token_table, segment_table):
    batch, s = sequence.shape
    vocab, d = token_table.shape
    n_slabs = 4
    slab_s = s // n_slabs
    slab_n = batch * slab_s

    pe = _pe_gen(s, d, bs=512)
    seg3 = segment_ids.reshape(batch, 1, s)

    out = None
    for j in range(n_slabs):
        lo = j * slab_s
        idx = sequence[:, lo : lo + slab_s].reshape(slab_n)
        tok = _sc_gather(token_table, idx, slab_n, d)
        out = _tc_add_slab(
            out, tok.reshape(batch, slab_s, d), seg3[:, :, lo : lo + slab_s],
            pe, segment_table, s, s_off=lo, bs=1024,
        )
    return out
